# baseline (speedup 1.0000x reference)
import functools

import jax
import jax.numpy as jnp
from jax import lax
from jax.experimental import pallas as pl
from jax.experimental.pallas import tpu as pltpu
from jax.experimental.pallas import tpu_sc as plsc


def _fuse_body(a_ref, w_ref, b_ref, o_ref):
    o_ref[...] = w_ref[...] + jnp.dot(
        a_ref[...], b_ref[...], preferred_element_type=jnp.float32
    )


def _fuse_table(weight, A, B, rows_per_block=1000):
    vocab, dim = weight.shape
    rank = A.shape[1]
    grid = pl.cdiv(vocab, rows_per_block)
    return pl.pallas_call(
        _fuse_body,
        grid=(grid,),
        in_specs=[
            pl.BlockSpec((rows_per_block, rank), lambda i: (i, 0)),
            pl.BlockSpec((rows_per_block, dim), lambda i: (i, 0)),
            pl.BlockSpec((rank, dim), lambda i: (0, 0)),
        ],
        out_specs=pl.BlockSpec((rows_per_block, dim), lambda i: (i, 0)),
        out_shape=jax.ShapeDtypeStruct((vocab, dim), jnp.float32),
    )(A, weight, B)


def _sc_gather(table, idx3d, hist):
    batch, _, hpad = idx3d.shape
    dim = table.shape[1]
    info = plsc.get_sparse_core_info()
    nc, ns = info.num_cores, info.num_subcores
    nw = nc * ns
    b_per_w = batch // nw
    mesh = plsc.VectorSubcoreMesh(core_axis_name="c", subcore_axis_name="s")

    @functools.partial(
        pl.kernel,
        mesh=mesh,
        compiler_params=pltpu.CompilerParams(use_tc_tiling_on_sc=False),
        out_type=jax.ShapeDtypeStruct((batch, hpad, 2 * dim), jnp.float32),
        scratch_types=[
            pltpu.VMEM((b_per_w, 2, hpad), jnp.int32),
            pltpu.VMEM((2, hpad, dim), jnp.float32),
            pltpu.SemaphoreType.DMA,
            pltpu.SemaphoreType.DMA,
        ],
    )
    def gather_kernel(table_hbm, idx_hbm, out_hbm, idx_v, rows_v, gsem, ssem):
        wid = lax.axis_index("s") * nc + lax.axis_index("c")
        base = wid * b_per_w
        pltpu.sync_copy(idx_hbm.at[pl.ds(base, b_per_w)], idx_v)

        def body(j, carry):
            lo = pltpu.async_copy(
                table_hbm.at[idx_v.at[j, 0]], rows_v.at[0], gsem
            )
            hi = pltpu.async_copy(
                table_hbm.at[idx_v.at[j, 1]], rows_v.at[1], gsem
            )
            lo.wait()
            hi.wait()
            slo = pltpu.async_copy(
                rows_v.at[0], out_hbm.at[base + j, :, pl.ds(0, dim)], ssem
            )
            shi = pltpu.async_copy(
                rows_v.at[1], out_hbm.at[base + j, :, pl.ds(dim, dim)], ssem
            )
            slo.wait()
            shi.wait()
            return carry

        lax.fori_loop(0, b_per_w, body, 0)

    return gather_kernel(table, idx3d)


def _xpose_body(x_ref, o_ref):
    x = x_ref[...]
    dim = o_ref.shape[1]
    hp = o_ref.shape[0] // 2
    for l in range(hp):
        xt = x[:, l, :].T
        o_ref[l] = xt[0:dim]
        o_ref[l + hp] = xt[dim : 2 * dim]


def _tc_transpose(packed, batch, hist, dim, bb=128):
    hpad = packed.shape[1]
    grid = (batch // bb,)
    out = pl.pallas_call(
        _xpose_body,
        grid=grid,
        in_specs=[pl.BlockSpec((bb, hpad, 2 * dim), lambda g: (g, 0, 0))],
        out_specs=pl.BlockSpec((hist, dim, bb), lambda g: (0, 0, g)),
        out_shape=jax.ShapeDtypeStruct((hist, dim, batch), jnp.float32),
    )(packed)
    return jnp.transpose(out, (2, 0, 1))


def kernel(indices, weight, A, B):
    batch, hist = indices.shape
    dim = weight.shape[1]
    hp = hist // 2
    hpad = hp + (-hp) % 8
    idx3d = jnp.pad(
        indices.astype(jnp.int32).reshape(batch, 2, hp),
        ((0, 0), (0, 0), (0, hpad - hp)),
    )
    fused = _fuse_table(weight, A, B)
    packed = _sc_gather(fused, idx3d, hist)
    return _tc_transpose(packed, batch, hist, dim)
